# baseline (speedup 1.0000x reference)
import functools

import jax
import jax.numpy as jnp
from jax import lax
from jax.experimental import pallas as pl
from jax.experimental.pallas import tpu as pltpu
from jax.experimental.pallas import tpu_sc as plsc

N = 10000
E = 640000
HID = 128
NG = 64
NPAD = 10240
NHALF = NPAD // 2
NLOC = NHALF + 128
NW = 32
CHUNK = 128
NCHUNK = 160
NB = 2
NBE = 4
LAG = NBE // 2
SUBCH = 40
RPT = NPAD // 16
LPT = NLOC // 16
DUMMY = N

_mesh = plsc.VectorSubcoreMesh(core_axis_name="c", subcore_axis_name="s")


@functools.partial(
    pl.kernel,
    mesh=_mesh,
    out_type=jax.ShapeDtypeStruct((2, NPAD), jnp.float32),
    scratch_types=[
        pltpu.VMEM((NCHUNK, CHUNK), jnp.int32),
        pltpu.VMEM((CHUNK,), jnp.float32),
        pltpu.VMEM((RPT,), jnp.float32),
        pltpu.VMEM_SHARED((NPAD,), jnp.float32),
        pltpu.SemaphoreType.DMA,
    ],
)
def _deg_kernel(dst_hbm, out_hbm, dst_v, ones_v, zeros_v, dacc_sh, sem):
    c = lax.axis_index("c")
    s = lax.axis_index("s")
    wid = s * 2 + c
    pltpu.sync_copy(dst_hbm.at[wid], dst_v)

    def _init(i, _):
        ones_v[pl.ds(i * 16, 16)] = jnp.ones((16,), jnp.float32)
        return 0
    lax.fori_loop(0, CHUNK // 16, _init, 0)

    def _zinit(i, _):
        zeros_v[pl.ds(i * 16, 16)] = jnp.zeros((16,), jnp.float32)
        return 0
    lax.fori_loop(0, RPT // 16, _zinit, 0)

    base = s * RPT
    pltpu.sync_copy(zeros_v, dacc_sh.at[pl.ds(base, RPT)])
    plsc.subcore_barrier()

    def _fire(j, _):
        pltpu.async_copy(ones_v, dacc_sh.at[dst_v.at[j]], sem, add=True)
        return 0
    lax.fori_loop(0, NCHUNK, _fire, 0)

    def _drain(j, _):
        pltpu.make_async_copy(ones_v, dacc_sh.at[dst_v.at[j]], sem).wait()
        return 0
    lax.fori_loop(0, NCHUNK, _drain, 0)

    plsc.subcore_barrier()
    pltpu.sync_copy(dacc_sh.at[pl.ds(base, RPT)], out_hbm.at[c, pl.ds(base, RPT)])


@functools.partial(
    pl.kernel,
    mesh=_mesh,
    out_type=jax.ShapeDtypeStruct((2, NPAD), jnp.float32),
    scratch_types=[
        pltpu.VMEM((NCHUNK + 8, CHUNK), jnp.int32),
        pltpu.VMEM((NCHUNK, CHUNK), jnp.int32),
        pltpu.VMEM((NCHUNK, CHUNK), jnp.float32),
        pltpu.VMEM((CHUNK,), jnp.float32),
        pltpu.VMEM((CHUNK,), jnp.float32),
        pltpu.VMEM((CHUNK,), jnp.float32),
        pltpu.VMEM((RPT,), jnp.float32),
        pltpu.VMEM_SHARED((NPAD,), jnp.float32),
        pltpu.SemaphoreType.DMA,
        pltpu.SemaphoreType.DMA,
    ],
)
def _s_kernel(src_hbm, dst_hbm, attr_hbm, r_hbm, out_hbm,
              src_v, dst_v, attr_v, rv0, rv1, svals, zeros_v, sacc_sh,
              sem0, sem1):
    c = lax.axis_index("c")
    s = lax.axis_index("s")
    wid = s * 2 + c
    pltpu.sync_copy(src_hbm.at[wid], src_v)
    pltpu.sync_copy(dst_hbm.at[wid], dst_v)
    pltpu.sync_copy(attr_hbm.at[wid], attr_v)

    def _zinit(i, _):
        zeros_v[pl.ds(i * 16, 16)] = jnp.zeros((16,), jnp.float32)
        return 0
    lax.fori_loop(0, RPT // 16, _zinit, 0)
    base = s * RPT
    pltpu.sync_copy(zeros_v, sacc_sh.at[pl.ds(base, RPT)])
    plsc.subcore_barrier()

    rvs = (rv0, rv1)
    sems = (sem0, sem1)
    for b in range(NB):
        pltpu.async_copy(r_hbm.at[src_v.at[b]], rvs[b], sems[b])

    def chunk_body(jj, _):
        for b in range(NB):
            j = jj * NB + b
            pltpu.make_async_copy(r_hbm.at[src_v.at[j]], rvs[b], sems[b]).wait()

            def _sv(u, _2):
                rv = rvs[b][pl.ds(u * 16, 16)]
                av = attr_v[j, pl.ds(u * 16, 16)]
                svals[pl.ds(u * 16, 16)] = rv * av
                return 0
            lax.fori_loop(0, CHUNK // 16, _sv, 0)
            pltpu.sync_copy(svals, sacc_sh.at[dst_v.at[j]], add=True)
            pltpu.async_copy(r_hbm.at[src_v.at[j + NB]], rvs[b], sems[b])
        return 0
    lax.fori_loop(0, NCHUNK // NB, chunk_body, 0)
    for b in range(NB):
        pltpu.make_async_copy(
            r_hbm.at[src_v.at[NCHUNK + b]], rvs[b], sems[b]).wait()

    plsc.subcore_barrier()
    pltpu.sync_copy(sacc_sh.at[pl.ds(base, RPT)], out_hbm.at[c, pl.ds(base, RPT)])


@functools.partial(
    pl.kernel,
    mesh=_mesh,
    out_type=jax.ShapeDtypeStruct((2, NLOC, HID), jnp.float32),
    scratch_types=[
        pltpu.VMEM((SUBCH + 8, CHUNK), jnp.int32),
        pltpu.VMEM((SUBCH, CHUNK), jnp.int32),
        pltpu.VMEM((CHUNK,), jnp.int32),
        pltpu.VMEM((CHUNK,), jnp.int32),
        pltpu.VMEM((CHUNK,), jnp.int32),
        pltpu.VMEM((CHUNK,), jnp.int32),
        pltpu.VMEM((CHUNK, HID), jnp.float32),
        pltpu.VMEM((CHUNK, HID), jnp.float32),
        pltpu.VMEM((CHUNK, HID), jnp.float32),
        pltpu.VMEM((CHUNK, HID), jnp.float32),
        pltpu.VMEM((8, HID), jnp.float32),
        pltpu.VMEM_SHARED((NLOC, HID), jnp.float32),
        pltpu.SemaphoreType.DMA,
        pltpu.SemaphoreType.DMA,
        pltpu.SemaphoreType.DMA,
        pltpu.SemaphoreType.DMA,
        pltpu.SemaphoreType.DMA,
        pltpu.SemaphoreType.DMA,
        pltpu.SemaphoreType.DMA,
        pltpu.SemaphoreType.DMA,
    ],
)
def _edge_kernel(hp_hbm, src_hbm, dst_hbm, acc_out,
                 src_v, dst_v, td0, td1, td2, td3, r0, r1, r2, r3, zrows,
                 acc_sh, g0, g1, g2, g3, s0, s1, s2, s3):
    c = lax.axis_index("c")
    s = lax.axis_index("s")
    lo = (c * NHALF).astype(jnp.int32)
    rows = (r0, r1, r2, r3)
    tds = (td0, td1, td2, td3)
    gsems = (g0, g1, g2, g3)
    ssems = (s0, s1, s2, s3)

    def _z(i, _):
        zrows[i // 8, pl.ds((i % 8) * 16, 16)] = jnp.zeros((16,), jnp.float32)
        return 0
    lax.fori_loop(0, 8 * 8, _z, 0)
    base = s * LPT
    for k in range(LPT // 8):
        pltpu.sync_copy(zrows, acc_sh.at[pl.ds(base + k * 8, 8)])
    plsc.subcore_barrier()

    def _gather(j, b):
        pltpu.async_copy(hp_hbm.at[src_v.at[j]], rows[b], gsems[b])

    def _gwait(j, b):
        pltpu.make_async_copy(hp_hbm.at[src_v.at[j]], rows[b], gsems[b]).wait()

    def _scat(b):
        pltpu.async_copy(rows[b], acc_sh.at[tds[b]], ssems[b], add=True)

    def _swait(b):
        pltpu.make_async_copy(rows[b], acc_sh.at[tds[b]], ssems[b]).wait()

    for slab_i in range(2):
        wid = s * 2 + slab_i
        for q in range(NCHUNK // SUBCH):
            pltpu.sync_copy(
                src_hbm.at[wid, pl.ds(q * SUBCH, SUBCH + 8)], src_v)
            pltpu.sync_copy(dst_hbm.at[wid, pl.ds(q * SUBCH, SUBCH)], dst_v)
            _gather(0, 0)
            _gather(1, 1)

            def chunk_body(jj, _):
                for b4 in range(4):
                    j = jj * 4 + b4
                    _gwait(j, b4)

                    def _reb(u, _2):
                        d = dst_v[j, pl.ds(u * 16, 16)] - lo
                        ok = (d >= 0) & (d < NHALF)
                        tds[b4][pl.ds(u * 16, 16)] = jnp.where(ok, d, NHALF)
                        return 0
                    lax.fori_loop(0, CHUNK // 16, _reb, 0)
                    bp = (b4 + 2) % 4

                    @pl.when(j >= 2)
                    def _ret():
                        _swait(bp)
                    _scat(b4)
                    _gather(j + 2, bp)
                return 0
            lax.fori_loop(0, SUBCH // 4, chunk_body, 0)
            for cc in range(SUBCH - 2, SUBCH):
                _swait(cc % 4)
            _gwait(SUBCH, SUBCH % 4)
            _gwait(SUBCH + 1, (SUBCH + 1) % 4)

    plsc.subcore_barrier()
    pltpu.sync_copy(acc_sh.at[pl.ds(base, LPT)], acc_out.at[c, pl.ds(base, LPT)])


_BLK = 1280
_GRID = NPAD // _BLK


def _row_spec(width):
    return pl.BlockSpec((_BLK, width), lambda i: (i, 0))


def _col_spec():
    return pl.BlockSpec((_BLK, 1), lambda i: (i, 0))


def _full_spec(shape):
    return pl.BlockSpec(shape, lambda i: tuple(0 for _ in shape))


def _prep_body(x_ref, degp_ref, e1w_ref, e1b_ref, e2w_ref, e2b_ref, c1w_ref,
               h_ref, hp_ref, r_ref, invd_ref):
    xb = x_ref[...]
    t1 = jnp.dot(xb, e1w_ref[...], preferred_element_type=jnp.float32) + e1b_ref[...]
    t2 = jnp.dot(t1, e2w_ref[...], preferred_element_type=jnp.float32) + e2b_ref[...]
    h = jnp.dot(t2, c1w_ref[...], preferred_element_type=jnp.float32)
    deg = jnp.sum(degp_ref[...], axis=1, keepdims=True) + 1.0
    r = lax.rsqrt(deg)
    h_ref[...] = h
    hp_ref[...] = h * r
    r_ref[...] = r
    invd_ref[...] = 1.0 / deg


def _prep(x, degp_t, e1w, e1b, e2w, e2b, c1w):
    return pl.pallas_call(
        _prep_body,
        grid=(_GRID,),
        in_specs=[_row_spec(128), pl.BlockSpec((_BLK, 2), lambda i: (i, 0)),
                  _full_spec((128, 128)), _full_spec((1, 128)),
                  _full_spec((128, 256)), _full_spec((1, 256)),
                  _full_spec((256, 128))],
        out_specs=[_row_spec(HID), _row_spec(HID), _col_spec(), _col_spec()],
        out_shape=[jax.ShapeDtypeStruct((NPAD, HID), jnp.float32),
                   jax.ShapeDtypeStruct((NPAD, HID), jnp.float32),
                   jax.ShapeDtypeStruct((NPAD, 1), jnp.float32),
                   jax.ShapeDtypeStruct((NPAD, 1), jnp.float32)],
    )(x, degp_t, e1w, e1b, e2w, e2b, c1w)


def _elu(x):
    return jnp.where(x > 0, x, jnp.exp(jnp.minimum(x, 0.0)) - 1.0)


def _combine1_body(acc_ref, h_ref, r_ref, invd_ref, sp_ref, we_ref,
                   b_ref, wn_ref, hn_ref, hpn_ref, rs_ref):
    r = r_ref[...]
    rs = r * jnp.sum(sp_ref[...], axis=1, keepdims=True)
    agg = (r * acc_ref[...] + rs * we_ref[...]
           + h_ref[...] * invd_ref[...] + b_ref[...])
    xn = _elu(agg)
    hn = jnp.dot(xn, wn_ref[...], preferred_element_type=jnp.float32)
    hn_ref[...] = hn
    hpn_ref[...] = hn * r
    rs_ref[...] = rs


def _combine1(acc, h, r, invd, sp_t, we, b, wn):
    return pl.pallas_call(
        _combine1_body,
        grid=(_GRID,),
        in_specs=[_row_spec(HID), _row_spec(HID), _col_spec(), _col_spec(),
                  pl.BlockSpec((_BLK, 2), lambda i: (i, 0)),
                  _full_spec((1, HID)), _full_spec((1, HID)),
                  _full_spec((HID, HID))],
        out_specs=[_row_spec(HID), _row_spec(HID), _col_spec()],
        out_shape=[jax.ShapeDtypeStruct((NPAD, HID), jnp.float32),
                   jax.ShapeDtypeStruct((NPAD, HID), jnp.float32),
                   jax.ShapeDtypeStruct((NPAD, 1), jnp.float32)],
    )(acc, h, r, invd, sp_t, we, b, wn)


def _combine_body(acc_ref, h_ref, r_ref, invd_ref, rs_ref, we_ref,
                  b_ref, wn_ref, hn_ref, hpn_ref):
    r = r_ref[...]
    agg = (r * acc_ref[...] + rs_ref[...] * we_ref[...]
           + h_ref[...] * invd_ref[...] + b_ref[...])
    xn = _elu(agg)
    hn = jnp.dot(xn, wn_ref[...], preferred_element_type=jnp.float32)
    hn_ref[...] = hn
    hpn_ref[...] = hn * r


def _combine(acc, h, r, invd, rs, we, b, wn):
    return pl.pallas_call(
        _combine_body,
        grid=(_GRID,),
        in_specs=[_row_spec(HID), _row_spec(HID),
                  _col_spec(), _col_spec(), _col_spec(),
                  _full_spec((1, HID)), _full_spec((1, HID)),
                  _full_spec((HID, HID))],
        out_specs=[_row_spec(HID), _row_spec(HID)],
        out_shape=[jax.ShapeDtypeStruct((NPAD, HID), jnp.float32),
                   jax.ShapeDtypeStruct((NPAD, HID), jnp.float32)],
    )(acc, h, r, invd, rs, we, b, wn)


def _final_body(acc_ref, h_ref, r_ref, invd_ref, rs_ref, we_ref, b_ref,
                batch_ref, lw_ref, lb_ref, out_ref, pooled, cnt):
    i = pl.program_id(0)

    @pl.when(i == 0)
    def _zero():
        pooled[...] = jnp.zeros_like(pooled)
        cnt[...] = jnp.zeros_like(cnt)

    r = r_ref[...]
    agg = (r * acc_ref[...] + rs_ref[...] * we_ref[...]
           + h_ref[...] * invd_ref[...] + b_ref[...])
    x4 = _elu(agg)
    oneh = (batch_ref[...] == lax.broadcasted_iota(jnp.int32, (_BLK, NG), 1)
            ).astype(jnp.float32)
    pooled[...] += lax.dot_general(oneh, x4, (((0,), (0,)), ((), ())),
                                   preferred_element_type=jnp.float32)
    cnt[...] += lax.dot_general(oneh, jnp.ones((_BLK, 1), jnp.float32),
                                (((0,), (0,)), ((), ())),
                                preferred_element_type=jnp.float32)

    @pl.when(i == _GRID - 1)
    def _emit():
        mean = pooled[...] / jnp.maximum(cnt[...], 1.0)
        out_ref[...] = (jnp.dot(mean, lw_ref[...],
                                preferred_element_type=jnp.float32)
                        + lb_ref[...])


def _final(acc, h, r, invd, rs, we, b, batch2d, lw, lb):
    return pl.pallas_call(
        _final_body,
        grid=(_GRID,),
        in_specs=[_row_spec(HID), _row_spec(HID),
                  _col_spec(), _col_spec(), _col_spec(),
                  _full_spec((1, HID)), _full_spec((1, HID)),
                  _col_spec(),
                  _full_spec((HID, 10)), _full_spec((1, 10))],
        out_specs=pl.BlockSpec((NG, 10), lambda i: (0, 0)),
        out_shape=jax.ShapeDtypeStruct((NG, 10), jnp.float32),
        scratch_shapes=[pltpu.VMEM((NG, HID), jnp.float32),
                        pltpu.VMEM((NG, 1), jnp.float32)],
    )(acc, h, r, invd, rs, we, b, batch2d, lw, lb)


def _acc_cat(a):
    return jnp.concatenate([a[0, :NHALF], a[1, :NHALF]], axis=0)


def kernel(x, edge_index, edge_attr, edge_type, batch,
           enc1_w, enc1_b, enc2_w, enc2_b,
           c1_w, c1_we, c1_b, c2_w, c2_we, c2_b,
           c3_w, c3_we, c3_b, c4_w, c4_we, c4_b,
           lin1_w, lin1_b):
    x = x.astype(jnp.float32)
    epad = NW * NCHUNK * CHUNK - E
    src = jnp.concatenate([edge_index[0], jnp.full((epad,), DUMMY, jnp.int32)])
    dst = jnp.concatenate([edge_index[1], jnp.full((epad,), DUMMY, jnp.int32)])
    src3 = src.reshape(NW, NCHUNK, CHUNK)
    src3 = jnp.concatenate(
        [src3, jnp.full((NW, 8, CHUNK), DUMMY, jnp.int32)], axis=1)
    dst3 = dst.reshape(NW, NCHUNK, CHUNK)
    attr3 = jnp.concatenate(
        [edge_attr[:, 0], jnp.zeros((epad,), jnp.float32)]).reshape(
            NW, NCHUNK, CHUNK)
    xp = jnp.pad(x, ((0, NPAD - N), (0, 0)))
    batch2d = jnp.pad(batch, (0, NPAD - N), constant_values=NG).reshape(NPAD, 1)

    degp = _deg_kernel(dst3)

    h1, hp1, r, invd = _prep(xp, degp.T, enc1_w, enc1_b.reshape(1, -1),
                             enc2_w, enc2_b.reshape(1, -1), c1_w)
    r1d = r.reshape(NPAD)

    sp = _s_kernel(src3, dst3, attr3, r1d)

    a = _edge_kernel(hp1, src3, dst3)
    h2, hp2, rs = _combine1(_acc_cat(a), h1, r, invd, sp.T, c1_we,
                            c1_b.reshape(1, -1), c2_w)
    a = _edge_kernel(hp2, src3, dst3)
    h3, hp3 = _combine(_acc_cat(a), h2, r, invd, rs, c2_we,
                       c2_b.reshape(1, -1), c3_w)
    a = _edge_kernel(hp3, src3, dst3)
    h4, hp4 = _combine(_acc_cat(a), h3, r, invd, rs, c3_we,
                       c3_b.reshape(1, -1), c4_w)
    a = _edge_kernel(hp4, src3, dst3)
    out = _final(_acc_cat(a), h4, r, invd, rs, c4_we, c4_b.reshape(1, -1),
                 batch2d, lin1_w, lin1_b.reshape(1, -1))
    return out


# speedup vs baseline: 1.1795x; 1.1795x over previous
import functools

import jax
import jax.numpy as jnp
from jax import lax
from jax.experimental import pallas as pl
from jax.experimental.pallas import tpu as pltpu
from jax.experimental.pallas import tpu_sc as plsc

N = 10000
E = 640000
HID = 128
NG = 64
NPAD = 10240
NHALF = NPAD // 2
NLOC = NHALF + 128
NW = 32
CHUNK = 128
NCHUNK = 160
NB = 2
NBE = 4
LAG = NBE // 2
SUBCH = 40
RPT = NPAD // 16
LPT = NLOC // 16
DUMMY = N

_mesh = plsc.VectorSubcoreMesh(core_axis_name="c", subcore_axis_name="s")


@functools.partial(
    pl.kernel,
    mesh=_mesh,
    out_type=jax.ShapeDtypeStruct((2, NPAD), jnp.float32),
    scratch_types=[
        pltpu.VMEM((NCHUNK, CHUNK), jnp.int32),
        pltpu.VMEM((CHUNK,), jnp.float32),
        pltpu.VMEM((RPT,), jnp.float32),
        pltpu.VMEM_SHARED((NPAD,), jnp.float32),
        pltpu.SemaphoreType.DMA,
    ],
)
def _deg_kernel(dst_hbm, out_hbm, dst_v, ones_v, zeros_v, dacc_sh, sem):
    c = lax.axis_index("c")
    s = lax.axis_index("s")
    wid = s * 2 + c
    pltpu.sync_copy(dst_hbm.at[wid], dst_v)

    def _init(i, _):
        ones_v[pl.ds(i * 16, 16)] = jnp.ones((16,), jnp.float32)
        return 0
    lax.fori_loop(0, CHUNK // 16, _init, 0)

    def _zinit(i, _):
        zeros_v[pl.ds(i * 16, 16)] = jnp.zeros((16,), jnp.float32)
        return 0
    lax.fori_loop(0, RPT // 16, _zinit, 0)

    base = s * RPT
    pltpu.sync_copy(zeros_v, dacc_sh.at[pl.ds(base, RPT)])
    plsc.subcore_barrier()

    def _fire(j, _):
        pltpu.async_copy(ones_v, dacc_sh.at[dst_v.at[j]], sem, add=True)
        return 0
    lax.fori_loop(0, NCHUNK, _fire, 0)

    def _drain(j, _):
        pltpu.make_async_copy(ones_v, dacc_sh.at[dst_v.at[j]], sem).wait()
        return 0
    lax.fori_loop(0, NCHUNK, _drain, 0)

    plsc.subcore_barrier()
    pltpu.sync_copy(dacc_sh.at[pl.ds(base, RPT)], out_hbm.at[c, pl.ds(base, RPT)])


@functools.partial(
    pl.kernel,
    mesh=_mesh,
    out_type=jax.ShapeDtypeStruct((2, NPAD), jnp.float32),
    scratch_types=[
        pltpu.VMEM((NCHUNK + 8, CHUNK), jnp.int32),
        pltpu.VMEM((NCHUNK, CHUNK), jnp.int32),
        pltpu.VMEM((NCHUNK, CHUNK), jnp.float32),
        pltpu.VMEM((CHUNK,), jnp.float32),
        pltpu.VMEM((CHUNK,), jnp.float32),
        pltpu.VMEM((CHUNK,), jnp.float32),
        pltpu.VMEM((RPT,), jnp.float32),
        pltpu.VMEM_SHARED((NPAD,), jnp.float32),
        pltpu.SemaphoreType.DMA,
        pltpu.SemaphoreType.DMA,
    ],
)
def _s_kernel(src_hbm, dst_hbm, attr_hbm, r_hbm, out_hbm,
              src_v, dst_v, attr_v, rv0, rv1, svals, zeros_v, sacc_sh,
              sem0, sem1):
    c = lax.axis_index("c")
    s = lax.axis_index("s")
    wid = s * 2 + c
    pltpu.sync_copy(src_hbm.at[wid], src_v)
    pltpu.sync_copy(dst_hbm.at[wid], dst_v)
    pltpu.sync_copy(attr_hbm.at[wid], attr_v)

    def _zinit(i, _):
        zeros_v[pl.ds(i * 16, 16)] = jnp.zeros((16,), jnp.float32)
        return 0
    lax.fori_loop(0, RPT // 16, _zinit, 0)
    base = s * RPT
    pltpu.sync_copy(zeros_v, sacc_sh.at[pl.ds(base, RPT)])
    plsc.subcore_barrier()

    rvs = (rv0, rv1)
    sems = (sem0, sem1)
    for b in range(NB):
        pltpu.async_copy(r_hbm.at[src_v.at[b]], rvs[b], sems[b])

    def chunk_body(jj, _):
        for b in range(NB):
            j = jj * NB + b
            pltpu.make_async_copy(r_hbm.at[src_v.at[j]], rvs[b], sems[b]).wait()

            def _sv(u, _2):
                rv = rvs[b][pl.ds(u * 16, 16)]
                av = attr_v[j, pl.ds(u * 16, 16)]
                svals[pl.ds(u * 16, 16)] = rv * av
                return 0
            lax.fori_loop(0, CHUNK // 16, _sv, 0)
            pltpu.sync_copy(svals, sacc_sh.at[dst_v.at[j]], add=True)
            pltpu.async_copy(r_hbm.at[src_v.at[j + NB]], rvs[b], sems[b])
        return 0
    lax.fori_loop(0, NCHUNK // NB, chunk_body, 0)
    for b in range(NB):
        pltpu.make_async_copy(
            r_hbm.at[src_v.at[NCHUNK + b]], rvs[b], sems[b]).wait()

    plsc.subcore_barrier()
    pltpu.sync_copy(sacc_sh.at[pl.ds(base, RPT)], out_hbm.at[c, pl.ds(base, RPT)])


@functools.partial(
    pl.kernel,
    mesh=_mesh,
    out_type=jax.ShapeDtypeStruct((2, NLOC, HID), jnp.float32),
    scratch_types=[
        pltpu.VMEM((SUBCH + 8, CHUNK), jnp.int32),
        pltpu.VMEM((SUBCH, CHUNK), jnp.int32),
        pltpu.VMEM((CHUNK, HID), jnp.float32),
        pltpu.VMEM((CHUNK, HID), jnp.float32),
        pltpu.VMEM((CHUNK, HID), jnp.float32),
        pltpu.VMEM((CHUNK, HID), jnp.float32),
        pltpu.VMEM((8, HID), jnp.float32),
        pltpu.VMEM_SHARED((NLOC, HID), jnp.float32),
        pltpu.SemaphoreType.DMA,
        pltpu.SemaphoreType.DMA,
        pltpu.SemaphoreType.DMA,
        pltpu.SemaphoreType.DMA,
        pltpu.SemaphoreType.DMA,
        pltpu.SemaphoreType.DMA,
        pltpu.SemaphoreType.DMA,
        pltpu.SemaphoreType.DMA,
    ],
)
def _edge_kernel(hp_hbm, src_hbm, rdst_hbm, acc_out,
                 src_v, rdst_v, r0, r1, r2, r3, zrows, acc_sh,
                 g0, g1, g2, g3, s0, s1, s2, s3):
    c = lax.axis_index("c")
    s = lax.axis_index("s")
    rows = (r0, r1, r2, r3)
    gsems = (g0, g1, g2, g3)
    ssems = (s0, s1, s2, s3)

    def _z(i, _):
        zrows[i // 8, pl.ds((i % 8) * 16, 16)] = jnp.zeros((16,), jnp.float32)
        return 0
    lax.fori_loop(0, 8 * 8, _z, 0)
    base = s * LPT
    for k in range(LPT // 8):
        pltpu.sync_copy(zrows, acc_sh.at[pl.ds(base + k * 8, 8)])
    plsc.subcore_barrier()

    def _gather(j, b):
        pltpu.async_copy(hp_hbm.at[src_v.at[j]], rows[b], gsems[b])

    def _gwait(j, b):
        pltpu.make_async_copy(hp_hbm.at[src_v.at[j]], rows[b], gsems[b]).wait()

    def _scat(j, b):
        pltpu.async_copy(rows[b], acc_sh.at[rdst_v.at[j]], ssems[b], add=True)

    def _swait(j, b):
        pltpu.make_async_copy(rows[b], acc_sh.at[rdst_v.at[j]],
                              ssems[b]).wait()

    for slab_i in range(2):
        wid = s * 2 + slab_i
        for q in range(NCHUNK // SUBCH):
            pltpu.sync_copy(
                src_hbm.at[wid, pl.ds(q * SUBCH, SUBCH + 8)], src_v)
            pltpu.sync_copy(
                rdst_hbm.at[c, wid, pl.ds(q * SUBCH, SUBCH)], rdst_v)
            _gather(0, 0)

            def chunk_body(jj, _):
                for b4 in range(4):
                    j = jj * 4 + b4
                    _gwait(j, b4)
                    bp = (b4 + 1) % 4

                    @pl.when(j >= 3)
                    def _ret():
                        _swait(j - 3, bp)
                    _scat(j, b4)
                    _gather(j + 1, bp)
                return 0
            lax.fori_loop(0, SUBCH // 4, chunk_body, 0)
            for cc in range(SUBCH - 3, SUBCH):
                _swait(cc, cc % 4)
            _gwait(SUBCH, SUBCH % 4)

    plsc.subcore_barrier()
    pltpu.sync_copy(acc_sh.at[pl.ds(base, LPT)], acc_out.at[c, pl.ds(base, LPT)])


def _rebase_body(d_ref, o_ref):
    d = d_ref[...]
    d1 = d - NHALF
    o_ref[0] = jnp.where((d >= 0) & (d < NHALF), d, NHALF)
    o_ref[1] = jnp.where((d1 >= 0) & (d1 < NHALF), d1, NHALF)


def _rebase_tc(dst2d):
    return pl.pallas_call(
        _rebase_body,
        grid=(8,),
        in_specs=[pl.BlockSpec((NW * NCHUNK // 8, CHUNK), lambda i: (i, 0))],
        out_specs=pl.BlockSpec((2, NW * NCHUNK // 8, CHUNK),
                               lambda i: (0, i, 0)),
        out_shape=jax.ShapeDtypeStruct((2, NW * NCHUNK, CHUNK), jnp.int32),
    )(dst2d)


_BLK = 1280
_GRID = NPAD // _BLK


def _row_spec(width):
    return pl.BlockSpec((_BLK, width), lambda i: (i, 0))


def _col_spec():
    return pl.BlockSpec((_BLK, 1), lambda i: (i, 0))


def _full_spec(shape):
    return pl.BlockSpec(shape, lambda i: tuple(0 for _ in shape))


def _prep_body(x_ref, degp_ref, e1w_ref, e1b_ref, e2w_ref, e2b_ref, c1w_ref,
               h_ref, hp_ref, r_ref, invd_ref):
    xb = x_ref[...]
    t1 = jnp.dot(xb, e1w_ref[...], preferred_element_type=jnp.float32) + e1b_ref[...]
    t2 = jnp.dot(t1, e2w_ref[...], preferred_element_type=jnp.float32) + e2b_ref[...]
    h = jnp.dot(t2, c1w_ref[...], preferred_element_type=jnp.float32)
    deg = jnp.sum(degp_ref[...], axis=1, keepdims=True) + 1.0
    r = lax.rsqrt(deg)
    h_ref[...] = h
    hp_ref[...] = h * r
    r_ref[...] = r
    invd_ref[...] = 1.0 / deg


def _prep(x, degp_t, e1w, e1b, e2w, e2b, c1w):
    return pl.pallas_call(
        _prep_body,
        grid=(_GRID,),
        in_specs=[_row_spec(128), pl.BlockSpec((_BLK, 2), lambda i: (i, 0)),
                  _full_spec((128, 128)), _full_spec((1, 128)),
                  _full_spec((128, 256)), _full_spec((1, 256)),
                  _full_spec((256, 128))],
        out_specs=[_row_spec(HID), _row_spec(HID), _col_spec(), _col_spec()],
        out_shape=[jax.ShapeDtypeStruct((NPAD, HID), jnp.float32),
                   jax.ShapeDtypeStruct((NPAD, HID), jnp.float32),
                   jax.ShapeDtypeStruct((NPAD, 1), jnp.float32),
                   jax.ShapeDtypeStruct((NPAD, 1), jnp.float32)],
    )(x, degp_t, e1w, e1b, e2w, e2b, c1w)


def _elu(x):
    return jnp.where(x > 0, x, jnp.exp(jnp.minimum(x, 0.0)) - 1.0)


def _combine1_body(acc_ref, h_ref, r_ref, invd_ref, sp_ref, we_ref,
                   b_ref, wn_ref, hn_ref, hpn_ref, rs_ref):
    r = r_ref[...]
    rs = r * jnp.sum(sp_ref[...], axis=1, keepdims=True)
    agg = (r * acc_ref[...] + rs * we_ref[...]
           + h_ref[...] * invd_ref[...] + b_ref[...])
    xn = _elu(agg)
    hn = jnp.dot(xn, wn_ref[...], preferred_element_type=jnp.float32)
    hn_ref[...] = hn
    hpn_ref[...] = hn * r
    rs_ref[...] = rs


def _combine1(acc, h, r, invd, sp_t, we, b, wn):
    return pl.pallas_call(
        _combine1_body,
        grid=(_GRID,),
        in_specs=[_row_spec(HID), _row_spec(HID), _col_spec(), _col_spec(),
                  pl.BlockSpec((_BLK, 2), lambda i: (i, 0)),
                  _full_spec((1, HID)), _full_spec((1, HID)),
                  _full_spec((HID, HID))],
        out_specs=[_row_spec(HID), _row_spec(HID), _col_spec()],
        out_shape=[jax.ShapeDtypeStruct((NPAD, HID), jnp.float32),
                   jax.ShapeDtypeStruct((NPAD, HID), jnp.float32),
                   jax.ShapeDtypeStruct((NPAD, 1), jnp.float32)],
    )(acc, h, r, invd, sp_t, we, b, wn)


def _combine_body(acc_ref, h_ref, r_ref, invd_ref, rs_ref, we_ref,
                  b_ref, wn_ref, hn_ref, hpn_ref):
    r = r_ref[...]
    agg = (r * acc_ref[...] + rs_ref[...] * we_ref[...]
           + h_ref[...] * invd_ref[...] + b_ref[...])
    xn = _elu(agg)
    hn = jnp.dot(xn, wn_ref[...], preferred_element_type=jnp.float32)
    hn_ref[...] = hn
    hpn_ref[...] = hn * r


def _combine(acc, h, r, invd, rs, we, b, wn):
    return pl.pallas_call(
        _combine_body,
        grid=(_GRID,),
        in_specs=[_row_spec(HID), _row_spec(HID),
                  _col_spec(), _col_spec(), _col_spec(),
                  _full_spec((1, HID)), _full_spec((1, HID)),
                  _full_spec((HID, HID))],
        out_specs=[_row_spec(HID), _row_spec(HID)],
        out_shape=[jax.ShapeDtypeStruct((NPAD, HID), jnp.float32),
                   jax.ShapeDtypeStruct((NPAD, HID), jnp.float32)],
    )(acc, h, r, invd, rs, we, b, wn)


def _final_body(acc_ref, h_ref, r_ref, invd_ref, rs_ref, we_ref, b_ref,
                batch_ref, lw_ref, lb_ref, out_ref, pooled, cnt):
    i = pl.program_id(0)

    @pl.when(i == 0)
    def _zero():
        pooled[...] = jnp.zeros_like(pooled)
        cnt[...] = jnp.zeros_like(cnt)

    r = r_ref[...]
    agg = (r * acc_ref[...] + rs_ref[...] * we_ref[...]
           + h_ref[...] * invd_ref[...] + b_ref[...])
    x4 = _elu(agg)
    oneh = (batch_ref[...] == lax.broadcasted_iota(jnp.int32, (_BLK, NG), 1)
            ).astype(jnp.float32)
    pooled[...] += lax.dot_general(oneh, x4, (((0,), (0,)), ((), ())),
                                   preferred_element_type=jnp.float32)
    cnt[...] += lax.dot_general(oneh, jnp.ones((_BLK, 1), jnp.float32),
                                (((0,), (0,)), ((), ())),
                                preferred_element_type=jnp.float32)

    @pl.when(i == _GRID - 1)
    def _emit():
        mean = pooled[...] / jnp.maximum(cnt[...], 1.0)
        out_ref[...] = (jnp.dot(mean, lw_ref[...],
                                preferred_element_type=jnp.float32)
                        + lb_ref[...])


def _final(acc, h, r, invd, rs, we, b, batch2d, lw, lb):
    return pl.pallas_call(
        _final_body,
        grid=(_GRID,),
        in_specs=[_row_spec(HID), _row_spec(HID),
                  _col_spec(), _col_spec(), _col_spec(),
                  _full_spec((1, HID)), _full_spec((1, HID)),
                  _col_spec(),
                  _full_spec((HID, 10)), _full_spec((1, 10))],
        out_specs=pl.BlockSpec((NG, 10), lambda i: (0, 0)),
        out_shape=jax.ShapeDtypeStruct((NG, 10), jnp.float32),
        scratch_shapes=[pltpu.VMEM((NG, HID), jnp.float32),
                        pltpu.VMEM((NG, 1), jnp.float32)],
    )(acc, h, r, invd, rs, we, b, batch2d, lw, lb)


def _acc_cat(a):
    return jnp.concatenate([a[0, :NHALF], a[1, :NHALF]], axis=0)


def kernel(x, edge_index, edge_attr, edge_type, batch,
           enc1_w, enc1_b, enc2_w, enc2_b,
           c1_w, c1_we, c1_b, c2_w, c2_we, c2_b,
           c3_w, c3_we, c3_b, c4_w, c4_we, c4_b,
           lin1_w, lin1_b):
    x = x.astype(jnp.float32)
    epad = NW * NCHUNK * CHUNK - E
    src = jnp.concatenate([edge_index[0], jnp.full((epad,), DUMMY, jnp.int32)])
    dst = jnp.concatenate([edge_index[1], jnp.full((epad,), DUMMY, jnp.int32)])
    src3 = src.reshape(NW, NCHUNK, CHUNK)
    src3 = jnp.concatenate(
        [src3, jnp.full((NW, 8, CHUNK), DUMMY, jnp.int32)], axis=1)
    dst3 = dst.reshape(NW, NCHUNK, CHUNK)
    attr3 = jnp.concatenate(
        [edge_attr[:, 0], jnp.zeros((epad,), jnp.float32)]).reshape(
            NW, NCHUNK, CHUNK)
    xp = jnp.pad(x, ((0, NPAD - N), (0, 0)))
    batch2d = jnp.pad(batch, (0, NPAD - N), constant_values=NG).reshape(NPAD, 1)

    degp = _deg_kernel(dst3)
    rdst = _rebase_tc(dst3.reshape(NW * NCHUNK, CHUNK)).reshape(
        2, NW, NCHUNK, CHUNK)

    h1, hp1, r, invd = _prep(xp, degp.T, enc1_w, enc1_b.reshape(1, -1),
                             enc2_w, enc2_b.reshape(1, -1), c1_w)
    r1d = r.reshape(NPAD)

    sp = _s_kernel(src3, dst3, attr3, r1d)

    a = _edge_kernel(hp1, src3, rdst)
    h2, hp2, rs = _combine1(_acc_cat(a), h1, r, invd, sp.T, c1_we,
                            c1_b.reshape(1, -1), c2_w)
    a = _edge_kernel(hp2, src3, rdst)
    h3, hp3 = _combine(_acc_cat(a), h2, r, invd, rs, c2_we,
                       c2_b.reshape(1, -1), c3_w)
    a = _edge_kernel(hp3, src3, rdst)
    h4, hp4 = _combine(_acc_cat(a), h3, r, invd, rs, c3_we,
                       c3_b.reshape(1, -1), c4_w)
    a = _edge_kernel(hp4, src3, rdst)
    out = _final(_acc_cat(a), h4, r, invd, rs, c4_we, c4_b.reshape(1, -1),
                 batch2d, lin1_w, lin1_b.reshape(1, -1))
    return out
